# baseline (speedup 1.0000x reference)
import functools

import jax
import jax.numpy as jnp
from jax import lax
from jax.experimental import pallas as pl
from jax.experimental.pallas import tpu as pltpu
from jax.experimental.pallas import tpu_sc as plsc

NC = 2
NS = 16
NW = NC * NS
EB = 128


def _sc_scatter_rows(d, rw, nrows_pad, gather, nbuf, phases, tc_tiling=False):
  rows_per_tile = nrows_pad // NS
  rwp = rw // phases
  mesh = plsc.VectorSubcoreMesh(core_axis_name="c", subcore_axis_name="s")
  scr = [
      pltpu.VMEM((rwp, EB), jnp.int32),
      pltpu.VMEM((rwp, EB), jnp.int32),
  ]
  scr += [pltpu.VMEM((EB, d), jnp.float32) for _ in range(nbuf)]
  scr += [pltpu.SemaphoreType.DMA for _ in range(2 * nbuf)]
  scr += [pltpu.VMEM_SHARED((nrows_pad, d), jnp.float32)]

  @functools.partial(
      pl.kernel,
      out_type=jax.ShapeDtypeStruct((NC, nrows_pad, d), jnp.float32),
      mesh=mesh,
      compiler_params=pltpu.CompilerParams(use_tc_tiling_on_sc=tc_tiling),
      scratch_types=scr,
  )
  def k(table_hbm, src_hbm, dst_hbm, zeros_hbm, out_hbm, srcb, dstb, *rest):
    rows = rest[:nbuf]
    gsem = rest[nbuf:2 * nbuf]
    ssem = rest[2 * nbuf:3 * nbuf]
    acc = rest[3 * nbuf]
    c = lax.axis_index("c")
    s = lax.axis_index("s")
    wid = c * NS + s
    r0 = s * rows_per_tile
    pltpu.sync_copy(zeros_hbm.at[pl.ds(r0, rows_per_tile)],
                    acc.at[pl.ds(r0, rows_per_tile)])
    if not gather:
      pltpu.sync_copy(table_hbm, rows[0])
    plsc.subcore_barrier()

    def gdesc(j, b):
      return pltpu.make_async_copy(table_hbm.at[srcb.at[j]], rows[b], gsem[b])

    def sdesc(j, b):
      src = rows[b] if gather else rows[0]
      return pltpu.make_async_copy(src, acc.at[dstb.at[j]], ssem[b])

    for ph in range(phases):
      er0 = wid * rw + ph * rwp
      if gather:
        pltpu.sync_copy(src_hbm.at[pl.ds(er0, rwp)], srcb)
      pltpu.sync_copy(dst_hbm.at[pl.ds(er0, rwp)], dstb)

      if gather:
        for b in range(nbuf):
          gdesc(b, b).start()
      else:
        for b in range(nbuf):
          sdesc(b, b).start(add=True)

      def grp(g, carry):
        base = g * nbuf
        if gather:
          for b in range(nbuf):
            j = base + b
            gdesc(j, b).wait()
            sdesc(j, b).start(add=True)
          for b in range(nbuf):
            j = base + b
            sdesc(j, b).wait()

            @pl.when(j + nbuf < rwp)
            def _(j=j, b=b):
              gdesc(j + nbuf, b).start()
        else:
          for b in range(nbuf):
            j = base + b
            sdesc(j, b).wait()

            @pl.when(j + nbuf < rwp)
            def _(j=j, b=b):
              sdesc(j + nbuf, b).start(add=True)
        return carry

      lax.fori_loop(0, rwp // nbuf, grp, 0)

    plsc.subcore_barrier()
    pltpu.sync_copy(acc.at[pl.ds(r0, rows_per_tile)],
                    out_hbm.at[c].at[pl.ds(r0, rows_per_tile)])

  return k


def _sc_agg_fsplit(dh, rw_tile, nrows_pad, nbuf):
  d = dh // 2
  rows_per_tile = nrows_pad // NS
  mesh = plsc.VectorSubcoreMesh(core_axis_name="c", subcore_axis_name="s")
  scr = [
      pltpu.VMEM((rw_tile, EB), jnp.int32),
      pltpu.VMEM((rw_tile, EB), jnp.int32),
  ]
  scr += [pltpu.VMEM((EB, d), jnp.float32) for _ in range(nbuf)]
  scr += [pltpu.SemaphoreType.DMA for _ in range(2 * nbuf)]
  scr += [pltpu.VMEM_SHARED((nrows_pad, d), jnp.float32)]

  @functools.partial(
      pl.kernel,
      out_type=jax.ShapeDtypeStruct((NC, nrows_pad, d), jnp.float32),
      mesh=mesh,
      compiler_params=pltpu.CompilerParams(use_tc_tiling_on_sc=False),
      scratch_types=scr,
  )
  def k(table_hbm, src_hbm, dst_hbm, zeros_hbm, out_hbm, srcb, dstb, *rest):
    rows = rest[:nbuf]
    gsem = rest[nbuf:2 * nbuf]
    ssem = rest[2 * nbuf:3 * nbuf]
    acc = rest[3 * nbuf]
    c = lax.axis_index("c")
    s = lax.axis_index("s")
    r0 = s * rows_per_tile
    pltpu.sync_copy(zeros_hbm.at[pl.ds(r0, rows_per_tile)],
                    acc.at[pl.ds(r0, rows_per_tile)])
    er0 = s * rw_tile
    pltpu.sync_copy(src_hbm.at[pl.ds(er0, rw_tile)], srcb)
    pltpu.sync_copy(dst_hbm.at[pl.ds(er0, rw_tile)], dstb)
    plsc.subcore_barrier()

    table_c = table_hbm.at[c]

    def gdesc(j, b):
      return pltpu.make_async_copy(table_c.at[srcb.at[j]], rows[b], gsem[b])

    def sdesc(j, b):
      return pltpu.make_async_copy(rows[b], acc.at[dstb.at[j]], ssem[b])

    for b in range(nbuf):
      gdesc(b, b).start()

    def grp(g, carry):
      base = g * nbuf
      for b in range(nbuf):
        j = base + b
        gdesc(j, b).wait()
        sdesc(j, b).start(add=True)
      for b in range(nbuf):
        j = base + b
        sdesc(j, b).wait()

        @pl.when(j + nbuf < rw_tile)
        def _(j=j, b=b):
          gdesc(j + nbuf, b).start()
      return carry

    lax.fori_loop(0, rw_tile // nbuf, grp, 0)
    plsc.subcore_barrier()
    pltpu.sync_copy(acc.at[pl.ds(r0, rows_per_tile)],
                    out_hbm.at[c].at[pl.ds(r0, rows_per_tile)])

  return k


def _tc_matmul(x, w1):
  m, din = x.shape
  dh = w1.shape[1]
  bm = 1000

  def body(x_ref, w_ref, h_ref):
    h_ref[:] = jnp.dot(x_ref[:], w_ref[:], preferred_element_type=jnp.float32)

  return pl.pallas_call(
      body,
      grid=(m // bm,),
      in_specs=[
          pl.BlockSpec((bm, din), lambda i: (i, 0)),
          pl.BlockSpec((din, dh), lambda i: (0, 0)),
      ],
      out_specs=pl.BlockSpec((bm, dh), lambda i: (i, 0)),
      out_shape=jax.ShapeDtypeStruct((m, dh), jnp.float32),
  )(x, w1)


def _tc_prescale(h, degp):
  m, dh = h.shape
  d = dh // 2
  bm = 1000

  def body(h_ref, dp_ref, g_ref, dis_ref):
    deg = dp_ref[0, :, 0:1] + dp_ref[1, :, 0:1] + 1.0
    dis = lax.rsqrt(deg)
    g = h_ref[:] * dis
    g_ref[0] = g[:, :d]
    g_ref[1] = g[:, d:]
    dis_ref[:] = dis

  return pl.pallas_call(
      body,
      grid=(m // bm,),
      in_specs=[
          pl.BlockSpec((bm, dh), lambda i: (i, 0)),
          pl.BlockSpec((NC, bm, 8), lambda i: (0, i, 0)),
      ],
      out_specs=[
          pl.BlockSpec((NC, bm, d), lambda i: (0, i, 0)),
          pl.BlockSpec((bm, 1), lambda i: (i, 0)),
      ],
      out_shape=[
          jax.ShapeDtypeStruct((NC, m, d), jnp.float32),
          jax.ShapeDtypeStruct((m, 1), jnp.float32),
      ],
  )(h, degp)


def _tc_mid(acc1, gsp, dis, b1, w2p):
  m = gsp.shape[1]
  d = gsp.shape[2]
  dh = 2 * d
  dp = w2p.shape[1]
  bm = 1000

  def body(a_ref, g_ref, dis_ref, b1_ref, w2_ref, g2_ref):
    dis = dis_ref[:]
    pre = jnp.concatenate(
        [(a_ref[0] + g_ref[0]) * dis, (a_ref[1] + g_ref[1]) * dis], axis=1)
    h = jnp.maximum(pre + b1_ref[:], 0.0)
    g2_ref[:] = jnp.dot(h, w2_ref[:], preferred_element_type=jnp.float32) * dis

  return pl.pallas_call(
      body,
      grid=(m // bm,),
      in_specs=[
          pl.BlockSpec((NC, bm, d), lambda i: (0, i, 0)),
          pl.BlockSpec((NC, bm, d), lambda i: (0, i, 0)),
          pl.BlockSpec((bm, 1), lambda i: (i, 0)),
          pl.BlockSpec((1, dh), lambda i: (0, 0)),
          pl.BlockSpec((dh, dp), lambda i: (0, 0)),
      ],
      out_specs=pl.BlockSpec((bm, dp), lambda i: (i, 0)),
      out_shape=jax.ShapeDtypeStruct((m, dp), jnp.float32),
  )(acc1, gsp, dis, b1, w2p)


def _tc_final(acc2, g2, dis, b2p, d_out):
  m, dp = g2.shape
  bm = 1000

  def body(a_ref, g2_ref, dis_ref, b2_ref, out_ref):
    full = (a_ref[0] + a_ref[1] + g2_ref[:]) * dis_ref[:] + b2_ref[:]
    out_ref[:] = full[:, :d_out]

  return pl.pallas_call(
      body,
      grid=(m // bm,),
      in_specs=[
          pl.BlockSpec((NC, bm, dp), lambda i: (0, i, 0)),
          pl.BlockSpec((bm, dp), lambda i: (i, 0)),
          pl.BlockSpec((bm, 1), lambda i: (i, 0)),
          pl.BlockSpec((1, dp), lambda i: (0, 0)),
      ],
      out_specs=pl.BlockSpec((bm, d_out), lambda i: (i, 0)),
      out_shape=jax.ShapeDtypeStruct((m, d_out), jnp.float32),
  )(acc2, g2, dis, b2p)


def kernel(x, edge_index, W1, b1, W2, b2):
  n, d_in = x.shape
  d_hid = W1.shape[1]
  d_out = W2.shape[1]
  e = edge_index.shape[1]

  step = NW * EB * 8
  e_pad = ((e + step - 1) // step) * step
  r = e_pad // EB
  rw = r // NW
  n_pad = ((n + 1 + NS * 8 - 1) // (NS * 8)) * (NS * 8)

  src = edge_index[0].astype(jnp.int32)
  dst = edge_index[1].astype(jnp.int32)
  src2d = jnp.concatenate(
      [src, jnp.zeros((e_pad - e,), jnp.int32)]).reshape(r, EB)
  dst2d = jnp.concatenate(
      [dst, jnp.full((e_pad - e,), n, jnp.int32)]).reshape(r, EB)

  zeros8 = jnp.zeros((n_pad, 8), jnp.float32)
  zeros64 = jnp.zeros((n_pad, d_hid // 2), jnp.float32)
  ones_blk = jnp.ones((EB, 8), jnp.float32)

  h1 = _tc_matmul(x, W1)

  deg_k = _sc_scatter_rows(8, rw, n_pad, gather=False, nbuf=4, phases=1)
  degp = deg_k(ones_blk, src2d, dst2d, zeros8)

  gsp, dis = _tc_prescale(h1, degp)

  rw1 = r // NS
  agg1_k = _sc_agg_fsplit(d_hid, rw1, n_pad, nbuf=5)
  acc1 = agg1_k(gsp, src2d, dst2d, zeros64)

  w2p = jnp.concatenate(
      [W2, jnp.zeros((d_hid, 8 - d_out), jnp.float32)], axis=1)
  b1r = b1.reshape(1, d_hid)
  g2 = _tc_mid(acc1, gsp, dis, b1r, w2p)

  agg2_k = _sc_scatter_rows(8, rw, n_pad, gather=True, nbuf=4, phases=1)
  acc2 = agg2_k(g2, src2d, dst2d, zeros8)

  b2p = jnp.concatenate(
      [b2, jnp.zeros((8 - d_out,), jnp.float32)]).reshape(1, 8)
  return _tc_final(acc2, g2, dis, b2p, d_out)


# speedup vs baseline: 1.0333x; 1.0333x over previous
import functools

import jax
import jax.numpy as jnp
from jax import lax
from jax.experimental import pallas as pl
from jax.experimental.pallas import tpu as pltpu
from jax.experimental.pallas import tpu_sc as plsc

NC = 2
NS = 16
NW = NC * NS
EB = 128


def _sc_scatter_rows(d, rw, nrows_pad, gather, nbuf, phases, tc_tiling=False):
  rows_per_tile = nrows_pad // NS
  rwp = rw // phases
  mesh = plsc.VectorSubcoreMesh(core_axis_name="c", subcore_axis_name="s")
  scr = [
      pltpu.VMEM((rwp, EB), jnp.int32),
      pltpu.VMEM((rwp, EB), jnp.int32),
  ]
  scr += [pltpu.VMEM((EB, d), jnp.float32) for _ in range(nbuf)]
  scr += [pltpu.SemaphoreType.DMA for _ in range(2 * nbuf)]
  scr += [pltpu.VMEM_SHARED((nrows_pad, d), jnp.float32)]

  @functools.partial(
      pl.kernel,
      out_type=jax.ShapeDtypeStruct((NC, nrows_pad, d), jnp.float32),
      mesh=mesh,
      compiler_params=pltpu.CompilerParams(use_tc_tiling_on_sc=tc_tiling),
      scratch_types=scr,
  )
  def k(table_hbm, src_hbm, dst_hbm, zeros_hbm, out_hbm, srcb, dstb, *rest):
    rows = rest[:nbuf]
    gsem = rest[nbuf:2 * nbuf]
    ssem = rest[2 * nbuf:3 * nbuf]
    acc = rest[3 * nbuf]
    c = lax.axis_index("c")
    s = lax.axis_index("s")
    wid = c * NS + s
    r0 = s * rows_per_tile
    pltpu.sync_copy(zeros_hbm.at[pl.ds(r0, rows_per_tile)],
                    acc.at[pl.ds(r0, rows_per_tile)])
    if not gather:
      pltpu.sync_copy(table_hbm, rows[0])
    plsc.subcore_barrier()

    def gdesc(j, b):
      return pltpu.make_async_copy(table_hbm.at[srcb.at[j]], rows[b], gsem[b])

    def sdesc(j, b):
      src = rows[b] if gather else rows[0]
      return pltpu.make_async_copy(src, acc.at[dstb.at[j]], ssem[b])

    for ph in range(phases):
      er0 = wid * rw + ph * rwp
      if gather:
        pltpu.sync_copy(src_hbm.at[pl.ds(er0, rwp)], srcb)
      pltpu.sync_copy(dst_hbm.at[pl.ds(er0, rwp)], dstb)

      if gather:
        for b in range(nbuf):
          gdesc(b, b).start()
      else:
        for b in range(nbuf):
          sdesc(b, b).start(add=True)

      def grp(g, carry):
        base = g * nbuf
        if gather:
          for b in range(nbuf):
            j = base + b
            gdesc(j, b).wait()
            sdesc(j, b).start(add=True)
          for b in range(nbuf):
            j = base + b
            sdesc(j, b).wait()

            @pl.when(j + nbuf < rwp)
            def _(j=j, b=b):
              gdesc(j + nbuf, b).start()
        else:
          for b in range(nbuf):
            j = base + b
            sdesc(j, b).wait()

            @pl.when(j + nbuf < rwp)
            def _(j=j, b=b):
              sdesc(j + nbuf, b).start(add=True)
        return carry

      lax.fori_loop(0, rwp // nbuf, grp, 0)

    plsc.subcore_barrier()
    pltpu.sync_copy(acc.at[pl.ds(r0, rows_per_tile)],
                    out_hbm.at[c].at[pl.ds(r0, rows_per_tile)])

  return k


def _sc_agg_fsplit(dh, rw_tile, nrows_pad, nbuf):
  d = dh // 2
  rows_per_tile = nrows_pad // NS
  mesh = plsc.VectorSubcoreMesh(core_axis_name="c", subcore_axis_name="s")
  scr = [
      pltpu.VMEM((rw_tile, EB), jnp.int32),
      pltpu.VMEM((rw_tile, EB), jnp.int32),
  ]
  scr += [pltpu.VMEM((EB, d), jnp.float32) for _ in range(nbuf)]
  scr += [pltpu.SemaphoreType.DMA for _ in range(2 * nbuf)]
  scr += [pltpu.VMEM_SHARED((nrows_pad, d), jnp.float32)]

  @functools.partial(
      pl.kernel,
      out_type=jax.ShapeDtypeStruct((NC, nrows_pad, d), jnp.float32),
      mesh=mesh,
      compiler_params=pltpu.CompilerParams(use_tc_tiling_on_sc=False),
      scratch_types=scr,
  )
  def k(table_hbm, src_hbm, dst_hbm, zeros_hbm, out_hbm, srcb, dstb, *rest):
    rows = rest[:nbuf]
    gsem = rest[nbuf:2 * nbuf]
    ssem = rest[2 * nbuf:3 * nbuf]
    acc = rest[3 * nbuf]
    c = lax.axis_index("c")
    s = lax.axis_index("s")
    r0 = s * rows_per_tile
    pltpu.sync_copy(zeros_hbm.at[pl.ds(r0, rows_per_tile)],
                    acc.at[pl.ds(r0, rows_per_tile)])
    er0 = s * rw_tile
    pltpu.sync_copy(src_hbm.at[pl.ds(er0, rw_tile)], srcb)
    pltpu.sync_copy(dst_hbm.at[pl.ds(er0, rw_tile)], dstb)
    plsc.subcore_barrier()

    table_c = table_hbm.at[c]

    def gdesc(j, b):
      return pltpu.make_async_copy(table_c.at[srcb.at[j]], rows[b], gsem[b])

    def sdesc(j, b):
      return pltpu.make_async_copy(rows[b], acc.at[dstb.at[j]], ssem[b])

    for b in range(nbuf):
      gdesc(b, b).start()

    def grp(g, carry):
      base = g * nbuf
      for b in range(nbuf):
        j = base + b
        gdesc(j, b).wait()
        sdesc(j, b).start(add=True)
      for b in range(nbuf):
        j = base + b
        sdesc(j, b).wait()

        @pl.when(j + nbuf < rw_tile)
        def _(j=j, b=b):
          gdesc(j + nbuf, b).start()
      return carry

    lax.fori_loop(0, rw_tile // nbuf, grp, 0)
    plsc.subcore_barrier()
    pltpu.sync_copy(acc.at[pl.ds(r0, rows_per_tile)],
                    out_hbm.at[c].at[pl.ds(r0, rows_per_tile)])

  return k


def _tc_matmul(x, w1):
  m, din = x.shape
  dh = w1.shape[1]
  bm = 1000

  def body(x_ref, w_ref, h_ref):
    h_ref[:] = jnp.dot(x_ref[:], w_ref[:], preferred_element_type=jnp.float32)

  return pl.pallas_call(
      body,
      grid=(m // bm,),
      in_specs=[
          pl.BlockSpec((bm, din), lambda i: (i, 0)),
          pl.BlockSpec((din, dh), lambda i: (0, 0)),
      ],
      out_specs=pl.BlockSpec((bm, dh), lambda i: (i, 0)),
      out_shape=jax.ShapeDtypeStruct((m, dh), jnp.float32),
  )(x, w1)


def _tc_prescale(h, degp):
  m, dh = h.shape
  d = dh // 2
  bm = 1000

  def body(h_ref, dp_ref, g_ref, dis_ref):
    deg = dp_ref[0, :, 0:1] + dp_ref[1, :, 0:1] + 1.0
    dis = lax.rsqrt(deg)
    g = h_ref[:] * dis
    g_ref[0] = g[:, :d]
    g_ref[1] = g[:, d:]
    dis_ref[:] = dis

  return pl.pallas_call(
      body,
      grid=(m // bm,),
      in_specs=[
          pl.BlockSpec((bm, dh), lambda i: (i, 0)),
          pl.BlockSpec((NC, bm, 16), lambda i: (0, i, 0)),
      ],
      out_specs=[
          pl.BlockSpec((NC, bm, d), lambda i: (0, i, 0)),
          pl.BlockSpec((bm, 1), lambda i: (i, 0)),
      ],
      out_shape=[
          jax.ShapeDtypeStruct((NC, m, d), jnp.float32),
          jax.ShapeDtypeStruct((m, 1), jnp.float32),
      ],
  )(h, degp)


def _tc_mid(acc1, gsp, dis, b1, w2p):
  m = gsp.shape[1]
  d = gsp.shape[2]
  dh = 2 * d
  dp = w2p.shape[1]
  bm = 1000

  def body(a_ref, g_ref, dis_ref, b1_ref, w2_ref, g2_ref):
    dis = dis_ref[:]
    pre = jnp.concatenate(
        [(a_ref[0] + g_ref[0]) * dis, (a_ref[1] + g_ref[1]) * dis], axis=1)
    h = jnp.maximum(pre + b1_ref[:], 0.0)
    g2_ref[:] = jnp.dot(h, w2_ref[:], preferred_element_type=jnp.float32) * dis

  return pl.pallas_call(
      body,
      grid=(m // bm,),
      in_specs=[
          pl.BlockSpec((NC, bm, d), lambda i: (0, i, 0)),
          pl.BlockSpec((NC, bm, d), lambda i: (0, i, 0)),
          pl.BlockSpec((bm, 1), lambda i: (i, 0)),
          pl.BlockSpec((1, dh), lambda i: (0, 0)),
          pl.BlockSpec((dh, dp), lambda i: (0, 0)),
      ],
      out_specs=pl.BlockSpec((bm, dp), lambda i: (i, 0)),
      out_shape=jax.ShapeDtypeStruct((m, dp), jnp.float32),
  )(acc1, gsp, dis, b1, w2p)


def _tc_final(acc2, g2, dis, b2p, d_out):
  m, dp = g2.shape
  bm = 1000

  def body(a_ref, g2_ref, dis_ref, b2_ref, out_ref):
    full = (a_ref[0] + a_ref[1] + g2_ref[:]) * dis_ref[:] + b2_ref[:]
    out_ref[:] = full[:, :d_out]

  return pl.pallas_call(
      body,
      grid=(m // bm,),
      in_specs=[
          pl.BlockSpec((NC, bm, dp), lambda i: (0, i, 0)),
          pl.BlockSpec((bm, dp), lambda i: (i, 0)),
          pl.BlockSpec((bm, 1), lambda i: (i, 0)),
          pl.BlockSpec((1, dp), lambda i: (0, 0)),
      ],
      out_specs=pl.BlockSpec((bm, d_out), lambda i: (i, 0)),
      out_shape=jax.ShapeDtypeStruct((m, d_out), jnp.float32),
  )(acc2, g2, dis, b2p)


def kernel(x, edge_index, W1, b1, W2, b2):
  n, d_in = x.shape
  d_hid = W1.shape[1]
  d_out = W2.shape[1]
  e = edge_index.shape[1]

  step = NW * EB * 8
  e_pad = ((e + step - 1) // step) * step
  r = e_pad // EB
  rw = r // NW
  n_pad = ((n + 1 + NS * 8 - 1) // (NS * 8)) * (NS * 8)

  src = edge_index[0].astype(jnp.int32)
  dst = edge_index[1].astype(jnp.int32)
  src2d = jnp.concatenate(
      [src, jnp.zeros((e_pad - e,), jnp.int32)]).reshape(r, EB)
  dst2d = jnp.concatenate(
      [dst, jnp.full((e_pad - e,), n, jnp.int32)]).reshape(r, EB)

  zeros8 = jnp.zeros((n_pad, 8), jnp.float32)
  zeros16 = jnp.zeros((n_pad, 16), jnp.float32)
  zeros64 = jnp.zeros((n_pad, d_hid // 2), jnp.float32)
  ones_blk = jnp.ones((EB, 16), jnp.float32)

  h1 = _tc_matmul(x, W1)

  deg_k = _sc_scatter_rows(16, rw, n_pad, gather=False, nbuf=4, phases=1)
  degp = deg_k(ones_blk, src2d, dst2d, zeros16)

  gsp, dis = _tc_prescale(h1, degp)

  rw1 = r // NS
  agg1_k = _sc_agg_fsplit(d_hid, rw1, n_pad, nbuf=5)
  acc1 = agg1_k(gsp, src2d, dst2d, zeros64)

  w2p = jnp.concatenate(
      [W2, jnp.zeros((d_hid, 8 - d_out), jnp.float32)], axis=1)
  b1r = b1.reshape(1, d_hid)
  g2 = _tc_mid(acc1, gsp, dis, b1r, w2p)

  agg2_k = _sc_scatter_rows(8, rw, n_pad, gather=True, nbuf=4, phases=1)
  acc2 = agg2_k(g2, src2d, dst2d, zeros8)

  b2p = jnp.concatenate(
      [b2, jnp.zeros((8 - d_out,), jnp.float32)]).reshape(1, 8)
  return _tc_final(acc2, g2, dis, b2p, d_out)
